# Initial kernel scaffold; baseline (speedup 1.0000x reference)
import jax
import jax.numpy as jnp
from jax.experimental import pallas as pl


def kernel(input_ids, row_pos_from, row_pos_to, col_pos_from, col_pos_to, row_table, col_table):
    raise NotImplementedError("write your pallas kernel here")


# speedup vs baseline: 1.4898x; 1.4898x over previous
import functools

import jax
import jax.numpy as jnp
from jax import lax
from jax.experimental import pallas as pl
from jax.experimental.pallas import tpu as pltpu
from jax.experimental.pallas import tpu_sc as plsc

DEPTH = 128
D = 768
TOTAL = 32 * 1024
NW = 32
RPW = TOTAL // NW
C = 32
NCHUNK = RPW // C
L = 16
MAGIC = 8388608.0


def _round_f32(x):
    return (x + MAGIC) - MAGIC


_mesh = plsc.VectorSubcoreMesh(
    core_axis_name="c", subcore_axis_name="s", num_cores=2, num_subcores=16
)


@functools.partial(
    pl.kernel,
    out_type=jax.ShapeDtypeStruct((TOTAL, D), jnp.float32),
    mesh=_mesh,
    scratch_types=[
        pltpu.VMEM((RPW,), jnp.float32),
        pltpu.VMEM((RPW,), jnp.float32),
        pltpu.VMEM((RPW,), jnp.int32),
        pltpu.VMEM((RPW,), jnp.int32),
        pltpu.VMEM((C, D), jnp.float32),
        pltpu.VMEM((C, D), jnp.float32),
        pltpu.VMEM((C, D), jnp.float32),
        pltpu.SemaphoreType.DMA,
        pltpu.SemaphoreType.DMA,
        pltpu.SemaphoreType.DMA,
    ],
)
def _sc_kernel(in_hbm, rf, rt, cf, ct, row_tab, col_tab, out_hbm,
               pa, pb, ridx, cidx, in_buf, r_buf, c_buf, sem0, sem1, sem2):
    wid = lax.axis_index("s") * 2 + lax.axis_index("c")
    base = wid * RPW

    def compute_idx(pfrom, pto, idx_ref):
        pltpu.sync_copy(pfrom.at[pl.ds(base, RPW)], pa)
        pltpu.sync_copy(pto.at[pl.ds(base, RPW)], pb)

        def body(i, carry):
            a = _round_f32(pa[pl.ds(i * L, L)] * DEPTH)
            b = _round_f32(pb[pl.ds(i * L, L)] * DEPTH)
            s = (a + b).astype(jnp.int32)
            m = s >> 1
            idx = jnp.minimum(m + ((s & 1) & (m & 1)), DEPTH - 1)
            idx_ref[pl.ds(i * L, L)] = idx
            return carry

        lax.fori_loop(0, RPW // L, body, 0)

    compute_idx(rf, rt, ridx)
    compute_idx(cf, ct, cidx)

    def chunk_body(g, carry):
        row0 = base + g * C
        cp_in = pltpu.async_copy(in_hbm.at[pl.ds(row0, C)], in_buf, sem0)
        cp_r = pltpu.async_copy(
            row_tab.at[ridx.at[pl.ds(g * C, C)]], r_buf, sem1)
        cp_c = pltpu.async_copy(
            col_tab.at[cidx.at[pl.ds(g * C, C)]], c_buf, sem2)
        cp_in.wait()
        cp_r.wait()
        cp_c.wait()

        def add_body(i, c2):
            for k in range(D // L):
                sl = pl.ds(k * L, L)
                plsc.addupdate(in_buf.at[i, sl], r_buf[i, sl] + c_buf[i, sl])
            return c2

        lax.fori_loop(0, C, add_body, 0)

        pltpu.sync_copy(in_buf, out_hbm.at[pl.ds(row0, C)])
        return carry

    lax.fori_loop(0, NCHUNK, chunk_body, 0)


def kernel(input_ids, row_pos_from, row_pos_to, col_pos_from, col_pos_to,
           row_table, col_table):
    b, p, d = input_ids.shape
    out = _sc_kernel(
        input_ids.reshape(b * p, d),
        row_pos_from.reshape(-1),
        row_pos_to.reshape(-1),
        col_pos_from.reshape(-1),
        col_pos_to.reshape(-1),
        row_table,
        col_table,
    )
    return out.reshape(b, p, d)
